# Initial kernel scaffold; baseline (speedup 1.0000x reference)
import jax
import jax.numpy as jnp
from jax.experimental import pallas as pl


def kernel(features, edge_index, edge_weight, W1, W2):
    raise NotImplementedError("write your pallas kernel here")


# speedup vs baseline: 2.2737x; 2.2737x over previous
import functools

import jax
import jax.numpy as jnp
from jax import lax
from jax.experimental import pallas as pl
from jax.experimental.pallas import tpu as pltpu
from jax.experimental.pallas import tpu_sc as plsc

N_NODES = 10000
N_PAD = 10240
ROWS_PER_TILE = N_PAD // 16
CHUNK = 1024
HALF = 256
SUB = 128
NSUB = CHUNK // SUB
NSUBH = HALF // SUB
NWORK = 32
LANES = 16


def _mm_kernel(x_ref, w_ref, o_ref):
    o_ref[...] = jnp.dot(x_ref[...], w_ref[...],
                         preferred_element_type=jnp.float32)


def _tc_matmul(x, w):
    n, d = x.shape
    dout = w.shape[1]
    bm = 1000
    return pl.pallas_call(
        _mm_kernel,
        grid=(n // bm,),
        in_specs=[pl.BlockSpec((bm, d), lambda i: (i, 0)),
                  pl.BlockSpec((d, dout), lambda i: (0, 0))],
        out_specs=pl.BlockSpec((bm, dout), lambda i: (i, 0)),
        out_shape=jax.ShapeDtypeStruct((n, dout), jnp.float32),
    )(x, w)


def _relu_mm_kernel(p_ref, w_ref, o_ref):
    h = jnp.maximum(p_ref[0] + p_ref[1], 0.0)
    o_ref[...] = jnp.dot(h, w_ref[...], preferred_element_type=jnp.float32)


def _tc_relu_matmul(p, w):
    d = p.shape[2]
    dout = w.shape[1]
    bm = 1000
    return pl.pallas_call(
        _relu_mm_kernel,
        grid=(N_NODES // bm,),
        in_specs=[pl.BlockSpec((2, bm, d), lambda i: (0, i, 0)),
                  pl.BlockSpec((d, dout), lambda i: (0, 0))],
        out_specs=pl.BlockSpec((bm, dout), lambda i: (i, 0)),
        out_shape=jax.ShapeDtypeStruct((N_NODES, dout), jnp.float32),
    )(p, w)


def _sum_kernel(p_ref, o_ref):
    o_ref[...] = (p_ref[0] + p_ref[1])[:, :o_ref.shape[1]]


def _tc_sum(p, dout):
    d = p.shape[2]
    bm = 1000
    return pl.pallas_call(
        _sum_kernel,
        grid=(N_NODES // bm,),
        in_specs=[pl.BlockSpec((2, bm, d), lambda i: (0, i, 0))],
        out_specs=pl.BlockSpec((bm, dout), lambda i: (i, 0)),
        out_shape=jax.ShapeDtypeStruct((N_NODES, dout), jnp.float32),
    )(p)


def _sc_spmm(table, src_r, dst_r, w_rep, d):
    e_pad = src_r.shape[0] * SUB
    per_worker = e_pad // NWORK
    n_chunks = per_worker // CHUNK
    mesh = plsc.VectorSubcoreMesh(core_axis_name="c", subcore_axis_name="s",
                                  num_cores=2, num_subcores=16)

    @functools.partial(
        pl.kernel,
        out_type=jax.ShapeDtypeStruct((2, N_PAD, d), jnp.float32),
        mesh=mesh,
        scratch_types=[
            pltpu.VMEM((NSUB, SUB), jnp.int32),
            pltpu.VMEM((NSUB, SUB), jnp.int32),
            pltpu.VMEM((HALF * LANES,), jnp.float32),
            pltpu.VMEM((HALF, d), jnp.float32),
            pltpu.VMEM_SHARED((N_PAD, d), jnp.float32),
            pltpu.SemaphoreType.DMA,
        ],
    )
    def sck(table_hbm, src_hbm, dst_hbm, w_hbm, out_hbm,
            src_v, dst_v, w_v, rows_v, acc, sem):
        c = lax.axis_index("c")
        s = lax.axis_index("s")
        zero = jnp.zeros((LANES,), jnp.float32)

        def zrow(i, carry):
            for j in range(d // LANES):
                rows_v[i, pl.ds(j * LANES, LANES)] = zero
            return carry
        lax.fori_loop(0, HALF, zrow, 0)

        base = ROWS_PER_TILE * s
        off = 0
        while off < ROWS_PER_TILE:
            n = min(HALF, ROWS_PER_TILE - off)
            pltpu.sync_copy(rows_v.at[pl.ds(0, n)], acc.at[pl.ds(base + off, n)])
            off += n
        plsc.subcore_barrier()

        def chunk_body(k, carry):
            ebase = pl.multiple_of((c * 16 + s) * per_worker + k * CHUNK, CHUNK)
            erow = pl.multiple_of(ebase // SUB, NSUB)
            pltpu.sync_copy(src_hbm.at[pl.ds(erow, NSUB)], src_v)
            pltpu.sync_copy(dst_hbm.at[pl.ds(erow, NSUB)], dst_v)
            for h in range(CHUNK // HALF):
                pltpu.sync_copy(
                    w_hbm.at[pl.ds((ebase + h * HALF) * LANES, HALF * LANES)],
                    w_v)
                cps = [pltpu.async_copy(table_hbm.at[src_v.at[h * NSUBH + j]],
                                        rows_v.at[pl.ds(j * SUB, SUB)], sem)
                       for j in range(NSUBH)]
                for cp in cps:
                    cp.wait()

                def escale(e, carry2):
                    wv = w_v[pl.ds(e * LANES, LANES)]
                    for j in range(d // LANES):
                        seg = rows_v[e, pl.ds(j * LANES, LANES)]
                        rows_v[e, pl.ds(j * LANES, LANES)] = seg * wv
                    return carry2
                lax.fori_loop(0, HALF, escale, 0)

                for j in range(NSUBH):
                    pltpu.sync_copy(rows_v.at[pl.ds(j * SUB, SUB)],
                                    acc.at[dst_v.at[h * NSUBH + j]], add=True)
            return carry
        lax.fori_loop(0, n_chunks, chunk_body, 0)
        plsc.subcore_barrier()
        pltpu.sync_copy(acc.at[pl.ds(base, ROWS_PER_TILE)],
                        out_hbm.at[c, pl.ds(base, ROWS_PER_TILE)])

    return sck(table, src_r, dst_r, w_rep)


def kernel(features, edge_index, edge_weight, W1, W2):
    src = edge_index[0].astype(jnp.int32)
    dst = edge_index[1].astype(jnp.int32)
    w = edge_weight.astype(jnp.float32)
    e = src.shape[0]
    e_pad = ((e + NWORK * CHUNK - 1) // (NWORK * CHUNK)) * (NWORK * CHUNK)
    pad = e_pad - e
    src_r = jnp.pad(src, (0, pad)).reshape(e_pad // SUB, SUB)
    dst_r = jnp.pad(dst, (0, pad)).reshape(e_pad // SUB, SUB)
    w_rep = jnp.broadcast_to(jnp.pad(w, (0, pad))[:, None],
                             (e_pad, LANES)).reshape(e_pad * LANES)

    dout = W2.shape[1]
    w2_p = jnp.pad(W2, ((0, 0), (0, 128 - dout)))

    xw = _tc_matmul(features, W1)
    p1 = _sc_spmm(xw, src_r, dst_r, w_rep, xw.shape[1])
    hw = _tc_relu_matmul(p1, w2_p)
    p2 = _sc_spmm(hw, src_r, dst_r, w_rep, hw.shape[1])
    return _tc_sum(p2, dout)


# speedup vs baseline: 2.8247x; 1.2423x over previous
import functools

import jax
import jax.numpy as jnp
from jax import lax
from jax.experimental import pallas as pl
from jax.experimental.pallas import tpu as pltpu
from jax.experimental.pallas import tpu_sc as plsc

N_NODES = 10000
N_PAD = 10240
ROWS_PER_TILE = N_PAD // 16
CHUNK = 1024
HALF = 128
SUB = 128
NSUB = CHUNK // SUB
NWORK = 32
LANES = 16


def _mm_kernel(x_ref, w_ref, o_ref):
    o_ref[...] = jnp.dot(x_ref[...], w_ref[...],
                         preferred_element_type=jnp.float32)


def _tc_matmul(x, w):
    n, d = x.shape
    dout = w.shape[1]
    bm = 1000
    return pl.pallas_call(
        _mm_kernel,
        grid=(n // bm,),
        in_specs=[pl.BlockSpec((bm, d), lambda i: (i, 0)),
                  pl.BlockSpec((d, dout), lambda i: (0, 0))],
        out_specs=pl.BlockSpec((bm, dout), lambda i: (i, 0)),
        out_shape=jax.ShapeDtypeStruct((n, dout), jnp.float32),
    )(x, w)


def _relu_mm_kernel(p_ref, w_ref, o_ref):
    h = jnp.maximum(p_ref[0] + p_ref[1], 0.0)
    o_ref[...] = jnp.dot(h, w_ref[...], preferred_element_type=jnp.float32)


def _tc_relu_matmul(p, w):
    d = p.shape[2]
    dout = w.shape[1]
    bm = 1000
    return pl.pallas_call(
        _relu_mm_kernel,
        grid=(N_NODES // bm,),
        in_specs=[pl.BlockSpec((2, bm, d), lambda i: (0, i, 0)),
                  pl.BlockSpec((d, dout), lambda i: (0, 0))],
        out_specs=pl.BlockSpec((bm, dout), lambda i: (i, 0)),
        out_shape=jax.ShapeDtypeStruct((N_NODES, dout), jnp.float32),
    )(p, w)


def _sum_kernel(p_ref, o_ref):
    o_ref[...] = (p_ref[0] + p_ref[1])[:, :o_ref.shape[1]]


def _tc_sum(p, dout):
    d = p.shape[2]
    bm = 1000
    return pl.pallas_call(
        _sum_kernel,
        grid=(N_NODES // bm,),
        in_specs=[pl.BlockSpec((2, bm, d), lambda i: (0, i, 0))],
        out_specs=pl.BlockSpec((bm, dout), lambda i: (i, 0)),
        out_shape=jax.ShapeDtypeStruct((N_NODES, dout), jnp.float32),
    )(p)


def _sc_spmm(table, src_r, dst_r, w_rep, d):
    e_pad = src_r.shape[0] * SUB
    per_worker = e_pad // NWORK
    n_chunks = per_worker // CHUNK
    mesh = plsc.VectorSubcoreMesh(core_axis_name="c", subcore_axis_name="s",
                                  num_cores=2, num_subcores=16)

    @functools.partial(
        pl.kernel,
        out_type=jax.ShapeDtypeStruct((2, N_PAD, d), jnp.float32),
        mesh=mesh,
        scratch_types=[
            pltpu.VMEM((NSUB, SUB), jnp.int32),
            pltpu.VMEM((NSUB, SUB), jnp.int32),
            pltpu.VMEM((2, HALF * LANES), jnp.float32),
            pltpu.VMEM((2, HALF, d), jnp.float32),
            pltpu.VMEM_SHARED((N_PAD, d), jnp.float32),
            pltpu.SemaphoreType.DMA,
            pltpu.SemaphoreType.DMA,
        ],
    )
    def sck(table_hbm, src_hbm, dst_hbm, w_hbm, out_hbm,
            src_v, dst_v, w_v, rows_v, acc, sg0, sg1):
        c = lax.axis_index("c")
        s = lax.axis_index("s")
        sgs = (sg0, sg1)
        zero = jnp.zeros((LANES,), jnp.float32)

        def zrow(i, carry):
            for j in range(d // LANES):
                rows_v[0, i, pl.ds(j * LANES, LANES)] = zero
            return carry
        lax.fori_loop(0, HALF, zrow, 0)

        base = ROWS_PER_TILE * s
        for t in range(ROWS_PER_TILE // HALF):
            pltpu.sync_copy(rows_v.at[0], acc.at[pl.ds(base + t * HALF, HALF)])
        plsc.subcore_barrier()

        wbase = (c * 16 + s) * per_worker
        wrow = pl.multiple_of(wbase // SUB, NSUB)
        wb16 = wbase * LANES

        pltpu.sync_copy(src_hbm.at[pl.ds(wrow, NSUB)], src_v)
        pltpu.async_copy(table_hbm.at[src_v.at[0]], rows_v.at[0], sg0)
        pltpu.async_copy(w_hbm.at[pl.ds(pl.multiple_of(wb16, 8), HALF * LANES)],
                         w_v.at[0], sg0)

        def chunk_body(k, carry):
            kr = pl.multiple_of(wrow + k * NSUB, NSUB)
            pltpu.sync_copy(dst_hbm.at[pl.ds(kr, NSUB)], dst_v)
            for j in range(NSUB):
                b = j % 2
                nb = 1 - b
                pltpu.make_async_copy(table_hbm.at[pl.ds(0, HALF)],
                                      rows_v.at[b], sgs[b]).wait()
                pltpu.make_async_copy(w_hbm.at[pl.ds(0, HALF * LANES)],
                                      w_v.at[b], sgs[b]).wait()
                woff = pl.multiple_of(
                    wb16 + (k * NSUB + j + 1) * HALF * LANES, 8)
                if j < NSUB - 1:
                    pltpu.async_copy(table_hbm.at[src_v.at[j + 1]],
                                     rows_v.at[nb], sgs[nb])
                    pltpu.async_copy(w_hbm.at[pl.ds(woff, HALF * LANES)],
                                     w_v.at[nb], sgs[nb])
                else:
                    @pl.when(k < n_chunks - 1)
                    def _fire_next_chunk():
                        pltpu.sync_copy(src_hbm.at[pl.ds(kr + NSUB, NSUB)],
                                        src_v)
                        pltpu.async_copy(table_hbm.at[src_v.at[0]],
                                         rows_v.at[nb], sgs[nb])
                        pltpu.async_copy(w_hbm.at[pl.ds(woff, HALF * LANES)],
                                         w_v.at[nb], sgs[nb])

                def escale(e, carry2):
                    wv = w_v[b, pl.ds(e * LANES, LANES)]
                    for jj in range(d // LANES):
                        seg = rows_v[b, e, pl.ds(jj * LANES, LANES)]
                        rows_v[b, e, pl.ds(jj * LANES, LANES)] = seg * wv
                    return carry2
                lax.fori_loop(0, HALF, escale, 0)

                pltpu.sync_copy(rows_v.at[b], acc.at[dst_v.at[j]], add=True)
            return carry
        lax.fori_loop(0, n_chunks, chunk_body, 0)
        plsc.subcore_barrier()
        pltpu.sync_copy(acc.at[pl.ds(base, ROWS_PER_TILE)],
                        out_hbm.at[c, pl.ds(base, ROWS_PER_TILE)])

    return sck(table, src_r, dst_r, w_rep)


def kernel(features, edge_index, edge_weight, W1, W2):
    src = edge_index[0].astype(jnp.int32)
    dst = edge_index[1].astype(jnp.int32)
    w = edge_weight.astype(jnp.float32)
    e = src.shape[0]
    e_pad = ((e + NWORK * CHUNK - 1) // (NWORK * CHUNK)) * (NWORK * CHUNK)
    pad = e_pad - e
    src_r = jnp.pad(src, (0, pad)).reshape(e_pad // SUB, SUB)
    dst_r = jnp.pad(dst, (0, pad)).reshape(e_pad // SUB, SUB)
    w_rep = jnp.broadcast_to(jnp.pad(w, (0, pad))[:, None],
                             (e_pad, LANES)).reshape(e_pad * LANES)

    dout = W2.shape[1]
    w2_p = jnp.pad(W2, ((0, 0), (0, 128 - dout)))

    xw = _tc_matmul(features, W1)
    p1 = _sc_spmm(xw, src_r, dst_r, w_rep, xw.shape[1])
    hw = _tc_relu_matmul(p1, w2_p)
    p2 = _sc_spmm(hw, src_r, dst_r, w_rep, hw.shape[1])
    return _tc_sum(p2, dout)


# speedup vs baseline: 2.8259x; 1.0004x over previous
import functools

import jax
import jax.numpy as jnp
from jax import lax
from jax.experimental import pallas as pl
from jax.experimental.pallas import tpu as pltpu
from jax.experimental.pallas import tpu_sc as plsc

N_NODES = 10000
N_PAD = 10240
ROWS_PER_TILE = N_PAD // 16
CHUNK = 1024
HALF = 128
SUB = 128
NSUB = CHUNK // SUB
NWORK = 32
LANES = 16


def _mm_kernel(x_ref, w_ref, o_ref):
    o_ref[...] = jnp.dot(x_ref[...], w_ref[...],
                         preferred_element_type=jnp.float32)


def _tc_matmul(x, w):
    n, d = x.shape
    dout = w.shape[1]
    bm = 1000
    return pl.pallas_call(
        _mm_kernel,
        grid=(n // bm,),
        in_specs=[pl.BlockSpec((bm, d), lambda i: (i, 0)),
                  pl.BlockSpec((d, dout), lambda i: (0, 0))],
        out_specs=pl.BlockSpec((bm, dout), lambda i: (i, 0)),
        out_shape=jax.ShapeDtypeStruct((n, dout), jnp.float32),
    )(x, w)


def _relu_mm_kernel(p_ref, w_ref, o_ref):
    h = jnp.maximum(p_ref[0] + p_ref[1], 0.0)
    o_ref[...] = jnp.dot(h, w_ref[...], preferred_element_type=jnp.float32)


def _tc_relu_matmul(p, w):
    d = p.shape[2]
    dout = w.shape[1]
    bm = 1000
    return pl.pallas_call(
        _relu_mm_kernel,
        grid=(N_NODES // bm,),
        in_specs=[pl.BlockSpec((2, bm, d), lambda i: (0, i, 0)),
                  pl.BlockSpec((d, dout), lambda i: (0, 0))],
        out_specs=pl.BlockSpec((bm, dout), lambda i: (i, 0)),
        out_shape=jax.ShapeDtypeStruct((N_NODES, dout), jnp.float32),
    )(p, w)


def _sum_kernel(p_ref, o_ref):
    o_ref[...] = (p_ref[0] + p_ref[1])[:, :o_ref.shape[1]]


def _tc_sum(p, dout):
    d = p.shape[2]
    bm = 1000
    return pl.pallas_call(
        _sum_kernel,
        grid=(N_NODES // bm,),
        in_specs=[pl.BlockSpec((2, bm, d), lambda i: (0, i, 0))],
        out_specs=pl.BlockSpec((bm, dout), lambda i: (i, 0)),
        out_shape=jax.ShapeDtypeStruct((N_NODES, dout), jnp.float32),
    )(p)


def _sc_spmm(table, src_r, dst_r, w_rep, d):
    e_pad = src_r.shape[0] * SUB
    per_worker = e_pad // NWORK
    n_chunks = per_worker // CHUNK
    mesh = plsc.VectorSubcoreMesh(core_axis_name="c", subcore_axis_name="s",
                                  num_cores=2, num_subcores=16)

    @functools.partial(
        pl.kernel,
        out_type=jax.ShapeDtypeStruct((2, N_PAD, d), jnp.float32),
        mesh=mesh,
        scratch_types=[
            pltpu.VMEM((NSUB, SUB), jnp.int32),
            pltpu.VMEM((NSUB, SUB), jnp.int32),
            pltpu.VMEM((2, HALF * LANES), jnp.float32),
            pltpu.VMEM((2, HALF, d), jnp.float32),
            pltpu.VMEM_SHARED((N_PAD, d), jnp.float32),
            pltpu.SemaphoreType.DMA,
            pltpu.SemaphoreType.DMA,
            pltpu.SemaphoreType.DMA,
            pltpu.SemaphoreType.DMA,
        ],
    )
    def sck(table_hbm, src_hbm, dst_hbm, w_hbm, out_hbm,
            src_v, dst_v, w_v, rows_v, acc, sg0, sg1, ss0, ss1):
        c = lax.axis_index("c")
        s = lax.axis_index("s")
        sgs = (sg0, sg1)
        sss = (ss0, ss1)
        zero = jnp.zeros((LANES,), jnp.float32)

        def zrow(i, carry):
            for j in range(d // LANES):
                rows_v[0, i, pl.ds(j * LANES, LANES)] = zero
            return carry
        lax.fori_loop(0, HALF, zrow, 0)

        base = ROWS_PER_TILE * s
        for t in range(ROWS_PER_TILE // HALF):
            pltpu.sync_copy(rows_v.at[0], acc.at[pl.ds(base + t * HALF, HALF)])
        plsc.subcore_barrier()

        wbase = (c * 16 + s) * per_worker
        wrow = pl.multiple_of(wbase // SUB, NSUB)
        wb16 = wbase * LANES

        pltpu.sync_copy(src_hbm.at[pl.ds(wrow, NSUB)], src_v)
        pltpu.async_copy(table_hbm.at[src_v.at[0]], rows_v.at[0], sg0)
        pltpu.async_copy(w_hbm.at[pl.ds(pl.multiple_of(wb16, 8), HALF * LANES)],
                         w_v.at[0], sg0)

        def chunk_body(k, carry):
            kr = pl.multiple_of(wrow + k * NSUB, NSUB)

            @pl.when(k > 0)
            def _drain_prev_scatter():
                pltpu.make_async_copy(table_hbm.at[pl.ds(0, HALF)],
                                      rows_v.at[1], sss[1]).wait()
            pltpu.sync_copy(dst_hbm.at[pl.ds(kr, NSUB)], dst_v)
            for j in range(NSUB):
                b = j % 2
                nb = 1 - b
                pltpu.make_async_copy(table_hbm.at[pl.ds(0, HALF)],
                                      rows_v.at[b], sgs[b]).wait()
                pltpu.make_async_copy(w_hbm.at[pl.ds(0, HALF * LANES)],
                                      w_v.at[b], sgs[b]).wait()
                if j > 0:
                    pltpu.make_async_copy(table_hbm.at[pl.ds(0, HALF)],
                                          rows_v.at[nb], sss[nb]).wait()
                woff = pl.multiple_of(
                    wb16 + (k * NSUB + j + 1) * HALF * LANES, 8)
                if j < NSUB - 1:
                    pltpu.async_copy(table_hbm.at[src_v.at[j + 1]],
                                     rows_v.at[nb], sgs[nb])
                    pltpu.async_copy(w_hbm.at[pl.ds(woff, HALF * LANES)],
                                     w_v.at[nb], sgs[nb])
                else:
                    @pl.when(k < n_chunks - 1)
                    def _fire_next_chunk():
                        pltpu.sync_copy(src_hbm.at[pl.ds(kr + NSUB, NSUB)],
                                        src_v)
                        pltpu.async_copy(table_hbm.at[src_v.at[0]],
                                         rows_v.at[nb], sgs[nb])
                        pltpu.async_copy(w_hbm.at[pl.ds(woff, HALF * LANES)],
                                         w_v.at[nb], sgs[nb])

                def escale(e, carry2):
                    wv = w_v[b, pl.ds(e * LANES, LANES)]
                    for jj in range(d // LANES):
                        seg = rows_v[b, e, pl.ds(jj * LANES, LANES)]
                        rows_v[b, e, pl.ds(jj * LANES, LANES)] = seg * wv
                    return carry2
                lax.fori_loop(0, HALF, escale, 0)

                pltpu.async_copy(rows_v.at[b], acc.at[dst_v.at[j]], sss[b],
                                 add=True)
            return carry
        lax.fori_loop(0, n_chunks, chunk_body, 0)
        pltpu.make_async_copy(table_hbm.at[pl.ds(0, HALF)],
                              rows_v.at[1], sss[1]).wait()
        plsc.subcore_barrier()
        pltpu.sync_copy(acc.at[pl.ds(base, ROWS_PER_TILE)],
                        out_hbm.at[c, pl.ds(base, ROWS_PER_TILE)])

    return sck(table, src_r, dst_r, w_rep)


def kernel(features, edge_index, edge_weight, W1, W2):
    src = edge_index[0].astype(jnp.int32)
    dst = edge_index[1].astype(jnp.int32)
    w = edge_weight.astype(jnp.float32)
    e = src.shape[0]
    e_pad = ((e + NWORK * CHUNK - 1) // (NWORK * CHUNK)) * (NWORK * CHUNK)
    pad = e_pad - e
    src_r = jnp.pad(src, (0, pad)).reshape(e_pad // SUB, SUB)
    dst_r = jnp.pad(dst, (0, pad)).reshape(e_pad // SUB, SUB)
    w_rep = jnp.broadcast_to(jnp.pad(w, (0, pad))[:, None],
                             (e_pad, LANES)).reshape(e_pad * LANES)

    dout = W2.shape[1]
    w2_p = jnp.pad(W2, ((0, 0), (0, 128 - dout)))

    xw = _tc_matmul(features, W1)
    p1 = _sc_spmm(xw, src_r, dst_r, w_rep, xw.shape[1])
    hw = _tc_relu_matmul(p1, w2_p)
    p2 = _sc_spmm(hw, src_r, dst_r, w_rep, hw.shape[1])
    return _tc_sum(p2, dout)


# speedup vs baseline: 2.8604x; 1.0122x over previous
import functools

import jax
import jax.numpy as jnp
from jax import lax
from jax.experimental import pallas as pl
from jax.experimental.pallas import tpu as pltpu
from jax.experimental.pallas import tpu_sc as plsc

N_NODES = 10000
N_PAD = 10240
ROWS_PER_TILE = N_PAD // 16
CHUNK = 1024
HALF = 128
SUB = 128
NSUB = CHUNK // SUB
NWORK = 32
LANES = 16


def _mm_kernel(x_ref, w_ref, o_ref):
    o_ref[...] = jnp.dot(x_ref[...], w_ref[...],
                         preferred_element_type=jnp.float32)


def _tc_matmul(x, w):
    n, d = x.shape
    dout = w.shape[1]
    bm = 1000
    return pl.pallas_call(
        _mm_kernel,
        grid=(n // bm,),
        in_specs=[pl.BlockSpec((bm, d), lambda i: (i, 0)),
                  pl.BlockSpec((d, dout), lambda i: (0, 0))],
        out_specs=pl.BlockSpec((bm, dout), lambda i: (i, 0)),
        out_shape=jax.ShapeDtypeStruct((n, dout), jnp.float32),
    )(x, w)


def _relu_mm_kernel(p_ref, w_ref, o_ref):
    h = jnp.maximum(p_ref[0] + p_ref[1], 0.0)
    o_ref[...] = jnp.dot(h, w_ref[...], preferred_element_type=jnp.float32)


def _tc_relu_matmul(p, w):
    d = p.shape[2]
    dout = w.shape[1]
    bm = 1000
    return pl.pallas_call(
        _relu_mm_kernel,
        grid=(N_NODES // bm,),
        in_specs=[pl.BlockSpec((2, bm, d), lambda i: (0, i, 0)),
                  pl.BlockSpec((d, dout), lambda i: (0, 0))],
        out_specs=pl.BlockSpec((bm, dout), lambda i: (i, 0)),
        out_shape=jax.ShapeDtypeStruct((N_NODES, dout), jnp.float32),
    )(p, w)


def _sum_kernel(p_ref, o_ref):
    o_ref[...] = (p_ref[0] + p_ref[1])[:, :o_ref.shape[1]]


def _tc_sum(p, dout):
    d = p.shape[2]
    bm = 1000
    return pl.pallas_call(
        _sum_kernel,
        grid=(N_NODES // bm,),
        in_specs=[pl.BlockSpec((2, bm, d), lambda i: (0, i, 0))],
        out_specs=pl.BlockSpec((bm, dout), lambda i: (i, 0)),
        out_shape=jax.ShapeDtypeStruct((N_NODES, dout), jnp.float32),
    )(p)


def _sc_spmm(table, src_r, dst_r, w_rep, d):
    e_pad = src_r.shape[0] * SUB
    per_worker = e_pad // NWORK
    n_chunks = per_worker // CHUNK
    mesh = plsc.VectorSubcoreMesh(core_axis_name="c", subcore_axis_name="s",
                                  num_cores=2, num_subcores=16)

    @functools.partial(
        pl.kernel,
        out_type=jax.ShapeDtypeStruct((2, N_PAD, d), jnp.float32),
        mesh=mesh,
        scratch_types=[
            pltpu.VMEM((NSUB, SUB), jnp.int32),
            pltpu.VMEM((NSUB, SUB), jnp.int32),
            pltpu.VMEM((2, HALF * LANES), jnp.float32),
            pltpu.VMEM((2, HALF, d), jnp.float32),
            pltpu.VMEM_SHARED((N_PAD, d), jnp.float32),
            pltpu.SemaphoreType.DMA,
            pltpu.SemaphoreType.DMA,
            pltpu.SemaphoreType.DMA,
            pltpu.SemaphoreType.DMA,
        ],
    )
    def sck(table_hbm, src_hbm, dst_hbm, w_hbm, out_hbm,
            src_v, dst_v, w_v, rows_v, acc, sg0, sg1, ss0, ss1):
        c = lax.axis_index("c")
        s = lax.axis_index("s")
        sgs = (sg0, sg1)
        sss = (ss0, ss1)
        zero = jnp.zeros((LANES,), jnp.float32)

        def zrow(i, carry):
            for j in range(d // LANES):
                rows_v[0, i, pl.ds(j * LANES, LANES)] = zero
            return carry
        lax.fori_loop(0, HALF, zrow, 0)

        base = ROWS_PER_TILE * s
        for t in range(ROWS_PER_TILE // HALF):
            pltpu.sync_copy(rows_v.at[0], acc.at[pl.ds(base + t * HALF, HALF)])
        plsc.subcore_barrier()

        wbase = (c * 16 + s) * per_worker
        wrow = pl.multiple_of(wbase // SUB, NSUB)
        wb16 = wbase * LANES

        pltpu.sync_copy(src_hbm.at[pl.ds(wrow, NSUB)], src_v)
        pltpu.async_copy(table_hbm.at[src_v.at[0]], rows_v.at[0], sg0)
        pltpu.async_copy(w_hbm.at[pl.ds(pl.multiple_of(wb16, 8), HALF * LANES)],
                         w_v.at[0], sg0)

        def chunk_body(k, carry):
            kr = pl.multiple_of(wrow + k * NSUB, NSUB)

            @pl.when(k > 0)
            def _drain_prev_scatter():
                pltpu.make_async_copy(table_hbm.at[pl.ds(0, HALF)],
                                      rows_v.at[1], sss[1]).wait()
            pltpu.sync_copy(dst_hbm.at[pl.ds(kr, NSUB)], dst_v)
            for j in range(NSUB):
                b = j % 2
                nb = 1 - b
                pltpu.make_async_copy(table_hbm.at[pl.ds(0, HALF)],
                                      rows_v.at[b], sgs[b]).wait()
                pltpu.make_async_copy(w_hbm.at[pl.ds(0, HALF * LANES)],
                                      w_v.at[b], sgs[b]).wait()
                if j > 0:
                    pltpu.make_async_copy(table_hbm.at[pl.ds(0, HALF)],
                                          rows_v.at[nb], sss[nb]).wait()
                woff = pl.multiple_of(
                    wb16 + (k * NSUB + j + 1) * HALF * LANES, 8)
                if j < NSUB - 1:
                    pltpu.async_copy(table_hbm.at[src_v.at[j + 1]],
                                     rows_v.at[nb], sgs[nb])
                    pltpu.async_copy(w_hbm.at[pl.ds(woff, HALF * LANES)],
                                     w_v.at[nb], sgs[nb])
                else:
                    @pl.when(k < n_chunks - 1)
                    def _fire_next_chunk():
                        pltpu.sync_copy(src_hbm.at[pl.ds(kr + NSUB, NSUB)],
                                        src_v)
                        pltpu.async_copy(table_hbm.at[src_v.at[0]],
                                         rows_v.at[nb], sgs[nb])
                        pltpu.async_copy(w_hbm.at[pl.ds(woff, HALF * LANES)],
                                         w_v.at[nb], sgs[nb])

                def escale(e, carry2):
                    wv = w_v[b, pl.ds(e * LANES, LANES)]
                    for jj in range(d // LANES):
                        seg = rows_v[b, e, pl.ds(jj * LANES, LANES)]
                        rows_v[b, e, pl.ds(jj * LANES, LANES)] = seg * wv
                    return carry2
                lax.fori_loop(0, HALF, escale, 0)

                pltpu.async_copy(rows_v.at[b], acc.at[dst_v.at[j]], sss[b],
                                 add=True)
            return carry
        lax.fori_loop(0, n_chunks, chunk_body, 0)
        pltpu.make_async_copy(table_hbm.at[pl.ds(0, HALF)],
                              rows_v.at[1], sss[1]).wait()
        plsc.subcore_barrier()
        pltpu.sync_copy(acc.at[pl.ds(base, ROWS_PER_TILE)],
                        out_hbm.at[c, pl.ds(base, ROWS_PER_TILE)])

    return sck(table, src_r, dst_r, w_rep)


def kernel(features, edge_index, edge_weight, W1, W2):
    src = edge_index[0].astype(jnp.int32)
    dst = edge_index[1].astype(jnp.int32)
    w = edge_weight.astype(jnp.float32)
    e = src.shape[0]
    e_pad = ((e + NWORK * CHUNK - 1) // (NWORK * CHUNK)) * (NWORK * CHUNK)
    pad = e_pad - e
    src_r = jnp.pad(src, (0, pad)).reshape(e_pad // SUB, SUB)
    pad_dst = N_NODES + (jnp.arange(pad, dtype=jnp.int32) % (N_PAD - N_NODES))
    dst_r = jnp.concatenate([dst, pad_dst]).reshape(e_pad // SUB, SUB)
    w_rep = jnp.broadcast_to(jnp.pad(w, (0, pad))[:, None],
                             (e_pad, LANES)).reshape(e_pad * LANES)

    dout = W2.shape[1]
    w2_p = jnp.pad(W2, ((0, 0), (0, 128 - dout)))

    xw = _tc_matmul(features, W1)
    p1 = _sc_spmm(xw, src_r, dst_r, w_rep, xw.shape[1])
    hw = _tc_relu_matmul(p1, w2_p)
    p2 = _sc_spmm(hw, src_r, dst_r, w_rep, hw.shape[1])
    return _tc_sum(p2, dout)


# speedup vs baseline: 5.0734x; 1.7736x over previous
import functools

import jax
import jax.numpy as jnp
from jax import lax
from jax.experimental import pallas as pl
from jax.experimental.pallas import tpu as pltpu
from jax.experimental.pallas import tpu_sc as plsc

N_NODES = 10000
N_PAD = 10240
ROWS_PER_TILE = N_PAD // 16
CHUNK = 1024
HALF = 128
SUB = 128
NSUB = CHUNK // SUB
NWORK = 32
LANES = 16


def _mm_kernel(x_ref, w_ref, o_ref):
    o_ref[...] = jnp.dot(x_ref[...], w_ref[...],
                         preferred_element_type=jnp.float32)


def _tc_matmul(x, w):
    n, d = x.shape
    dout = w.shape[1]
    bm = 1000
    return pl.pallas_call(
        _mm_kernel,
        grid=(n // bm,),
        in_specs=[pl.BlockSpec((bm, d), lambda i: (i, 0)),
                  pl.BlockSpec((d, dout), lambda i: (0, 0))],
        out_specs=pl.BlockSpec((bm, dout), lambda i: (i, 0)),
        out_shape=jax.ShapeDtypeStruct((n, dout), jnp.float32),
    )(x, w)


def _relu_mm_kernel(p_ref, w_ref, o_ref):
    h = jnp.maximum(p_ref[0] + p_ref[1], 0.0)
    o_ref[...] = jnp.dot(h, w_ref[...], preferred_element_type=jnp.float32)


def _tc_relu_matmul(p, w):
    d = p.shape[2]
    dout = w.shape[1]
    bm = 1000
    return pl.pallas_call(
        _relu_mm_kernel,
        grid=(N_NODES // bm,),
        in_specs=[pl.BlockSpec((2, bm, d), lambda i: (0, i, 0)),
                  pl.BlockSpec((d, dout), lambda i: (0, 0))],
        out_specs=pl.BlockSpec((bm, dout), lambda i: (i, 0)),
        out_shape=jax.ShapeDtypeStruct((N_NODES, dout), jnp.float32),
    )(p, w)


def _sum_kernel(p_ref, o_ref):
    o_ref[...] = (p_ref[0] + p_ref[1])[:, :o_ref.shape[1]]


def _tc_sum(p, dout):
    d = p.shape[2]
    bm = 1000
    return pl.pallas_call(
        _sum_kernel,
        grid=(N_NODES // bm,),
        in_specs=[pl.BlockSpec((2, bm, d), lambda i: (0, i, 0))],
        out_specs=pl.BlockSpec((bm, dout), lambda i: (i, 0)),
        out_shape=jax.ShapeDtypeStruct((N_NODES, dout), jnp.float32),
    )(p)


def _sc_spmm(table, src_r, dst_r, w_rep, d):
    e_pad = src_r.shape[0] * SUB
    per_worker = e_pad // NWORK
    n_chunks = per_worker // CHUNK
    mesh = plsc.VectorSubcoreMesh(core_axis_name="c", subcore_axis_name="s",
                                  num_cores=2, num_subcores=16)

    @functools.partial(
        pl.kernel,
        out_type=jax.ShapeDtypeStruct((2, N_PAD, d), jnp.float32),
        mesh=mesh,
        scratch_types=[
            pltpu.VMEM((NSUB, SUB), jnp.int32),
            pltpu.VMEM((NSUB, SUB), jnp.int32),
            pltpu.VMEM((2, HALF * LANES), jnp.float32),
            pltpu.VMEM((2, HALF, d), jnp.float32),
            pltpu.VMEM_SHARED((N_PAD, d), jnp.float32),
            pltpu.SemaphoreType.DMA,
            pltpu.SemaphoreType.DMA,
            pltpu.SemaphoreType.DMA,
            pltpu.SemaphoreType.DMA,
        ],
    )
    def sck(table_hbm, src_hbm, dst_hbm, w_hbm, out_hbm,
            src_v, dst_v, w_v, rows_v, acc, sg0, sg1, ss0, ss1):
        c = lax.axis_index("c")
        s = lax.axis_index("s")
        sgs = (sg0, sg1)
        sss = (ss0, ss1)
        zero = jnp.zeros((LANES,), jnp.float32)

        def zrow(i, carry):
            for j in range(d // LANES):
                rows_v[0, i, pl.ds(j * LANES, LANES)] = zero
            return carry
        lax.fori_loop(0, HALF, zrow, 0)

        base = ROWS_PER_TILE * s
        for t in range(ROWS_PER_TILE // HALF):
            pltpu.sync_copy(rows_v.at[0], acc.at[pl.ds(base + t * HALF, HALF)])
        plsc.subcore_barrier()

        wbase = (c * 16 + s) * per_worker
        wrow = pl.multiple_of(wbase // SUB, NSUB)
        wb16 = wbase * LANES

        pltpu.sync_copy(src_hbm.at[pl.ds(wrow, NSUB)], src_v)
        pltpu.async_copy(table_hbm.at[src_v.at[0]], rows_v.at[0], sg0)
        pltpu.async_copy(w_hbm.at[pl.ds(pl.multiple_of(wb16, 8), HALF * LANES)],
                         w_v.at[0], sg0)

        def chunk_body(k, carry):
            kr = pl.multiple_of(wrow + k * NSUB, NSUB)

            @pl.when(k > 0)
            def _drain_prev_scatter():
                pltpu.make_async_copy(table_hbm.at[pl.ds(0, HALF)],
                                      rows_v.at[1], sss[1]).wait()
            pltpu.sync_copy(dst_hbm.at[pl.ds(kr, NSUB)], dst_v)
            for j in range(NSUB):
                b = j % 2
                nb = 1 - b
                pltpu.make_async_copy(table_hbm.at[pl.ds(0, HALF)],
                                      rows_v.at[b], sgs[b]).wait()
                pltpu.make_async_copy(w_hbm.at[pl.ds(0, HALF * LANES)],
                                      w_v.at[b], sgs[b]).wait()
                if j > 0:
                    pltpu.make_async_copy(table_hbm.at[pl.ds(0, HALF)],
                                          rows_v.at[nb], sss[nb]).wait()
                woff = pl.multiple_of(
                    wb16 + (k * NSUB + j + 1) * HALF * LANES, 8)
                if j < NSUB - 1:
                    pltpu.async_copy(table_hbm.at[src_v.at[j + 1]],
                                     rows_v.at[nb], sgs[nb])
                    pltpu.async_copy(w_hbm.at[pl.ds(woff, HALF * LANES)],
                                     w_v.at[nb], sgs[nb])
                else:
                    @pl.when(k < n_chunks - 1)
                    def _fire_next_chunk():
                        pltpu.sync_copy(src_hbm.at[pl.ds(kr + NSUB, NSUB)],
                                        src_v)
                        pltpu.async_copy(table_hbm.at[src_v.at[0]],
                                         rows_v.at[nb], sgs[nb])
                        pltpu.async_copy(w_hbm.at[pl.ds(woff, HALF * LANES)],
                                         w_v.at[nb], sgs[nb])

                def escale(e, carry2):
                    wv = w_v[b, pl.ds(e * LANES, LANES)]
                    for jj in range(d // LANES):
                        seg = rows_v[b, e, pl.ds(jj * LANES, LANES)]
                        rows_v[b, e, pl.ds(jj * LANES, LANES)] = seg * wv
                    return carry2
                lax.fori_loop(0, HALF, escale, 0)

                pltpu.async_copy(rows_v.at[b], acc.at[dst_v.at[j]], sss[b],
                                 add=True)
            return carry
        lax.fori_loop(0, n_chunks, chunk_body, 0)
        pltpu.make_async_copy(table_hbm.at[pl.ds(0, HALF)],
                              rows_v.at[1], sss[1]).wait()
        plsc.subcore_barrier()
        pltpu.sync_copy(acc.at[pl.ds(base, ROWS_PER_TILE)],
                        out_hbm.at[c, pl.ds(base, ROWS_PER_TILE)])

    return sck(table, src_r, dst_r, w_rep)


def kernel(features, edge_index, edge_weight, W1, W2):
    src = edge_index[0].astype(jnp.int32)
    dst = edge_index[1].astype(jnp.int32)
    w = edge_weight.astype(jnp.float32)
    e = src.shape[0]
    e_pad = ((e + NWORK * CHUNK - 1) // (NWORK * CHUNK)) * (NWORK * CHUNK)
    pad = e_pad - e
    pad_src = jnp.arange(pad, dtype=jnp.int32) % N_NODES
    src_r = jnp.concatenate([src, pad_src]).reshape(e_pad // SUB, SUB)
    pad_dst = N_NODES + (jnp.arange(pad, dtype=jnp.int32) % (N_PAD - N_NODES))
    dst_r = jnp.concatenate([dst, pad_dst]).reshape(e_pad // SUB, SUB)
    w_rep = jnp.broadcast_to(jnp.pad(w, (0, pad))[:, None],
                             (e_pad, LANES)).reshape(e_pad * LANES)

    dout = W2.shape[1]
    w2_p = jnp.pad(W2, ((0, 0), (0, 128 - dout)))

    xw = _tc_matmul(features, W1)
    p1 = _sc_spmm(xw, src_r, dst_r, w_rep, xw.shape[1])
    hw = _tc_relu_matmul(p1, w2_p)
    p2 = _sc_spmm(hw, src_r, dst_r, w_rep, hw.shape[1])
    return _tc_sum(p2, dout)
